# baseline (speedup 1.0000x reference)
import jax
import jax.numpy as jnp
import numpy as np
from jax import lax
from jax.experimental import pallas as pl
from jax.experimental.pallas import tpu as pltpu
from jax.experimental.pallas import tpu_sc as plsc

N_AGENTS = 16
N_ACTIONS = 8
SAMPLE_SIZE = 8
STATE_DIM = 64
EMBED = 64
B_TOTAL = 1024
BB = 16
GRID = B_TOTAL // BB
R = BB * SAMPLE_SIZE
ROWS = R * N_AGENTS
SJ = SAMPLE_SIZE * N_AGENTS
RTOT = B_TOTAL * SAMPLE_SIZE
NW = 32
RPW = RTOT // NW
WORDS_PW = RPW * N_AGENTS
OUT_PW = RPW * 128

_TRIL = ((np.arange(SJ)[None, :] // N_ACTIONS) <=
         np.arange(N_AGENTS)[:, None]).astype(np.float32)
_REPB = ((np.arange(ROWS)[:, None] // SJ) ==
         np.arange(BB)[None, :]).astype(np.float32)


def _sc_gather_body(aq_hbm, idx_hbm, out_hbm, aq_v, idx_v, out_v):
    wid = lax.axis_index("s") * 2 + lax.axis_index("c")
    pltpu.sync_copy(aq_hbm.at[pl.ds(wid * WORDS_PW, WORDS_PW)], aq_v)
    pltpu.sync_copy(idx_hbm.at[pl.ds(wid * WORDS_PW, WORDS_PW)], idx_v)

    def body(r, carry):
        base = idx_v[pl.ds(r * N_AGENTS, N_AGENTS)]
        col = r * 128 + lax.iota(jnp.int32, N_AGENTS) * N_ACTIONS
        for c in range(N_ACTIONS):
            vals = plsc.load_gather(aq_v, (base + c,))
            plsc.store_scatter(out_v, (col + c,), vals)
        return carry

    lax.fori_loop(0, RPW, body, 0)
    pltpu.sync_copy(out_v, out_hbm.at[pl.ds(wid * OUT_PW, OUT_PW)])


def _sc_gather(aq_flat, widx):
    mesh = plsc.VectorSubcoreMesh(core_axis_name="c", subcore_axis_name="s",
                                  num_cores=2, num_subcores=16)
    fn = pl.kernel(
        _sc_gather_body,
        out_type=jax.ShapeDtypeStruct((RTOT * 128,), jnp.float32),
        mesh=mesh,
        compiler_params=pltpu.CompilerParams(needs_layout_passes=False),
        scratch_types=[
            pltpu.VMEM((WORDS_PW,), jnp.float32),
            pltpu.VMEM((WORDS_PW,), jnp.int32),
            pltpu.VMEM((OUT_PW,), jnp.float32),
        ],
    )
    return fn(aq_flat, widx)


def _mixer_block(pf_ref, gl_ref, st_ref, tril_ref, repb_ref,
                 w1s_ref, w1a_ref, b1_ref, w2_ref, b2_ref, w3_ref, b3_ref,
                 vw1_ref, vb1_ref, vw2_ref, vb2_ref, out_ref):
    f32 = jnp.float32
    pflat3 = pf_ref[:]
    gl = gl_ref[:]
    st = st_ref[:]

    xmask = (pflat3[:, :, None, :] *
             tril_ref[:][None, None]).reshape(ROWS, SJ)

    sp = jnp.dot(st, w1s_ref[:], preferred_element_type=f32)
    h1 = jnp.maximum(
        jnp.dot(xmask, w1a_ref[:], preferred_element_type=f32)
        + jnp.dot(repb_ref[:], sp, preferred_element_type=f32)
        + b1_ref[:], 0.0)
    h2 = jnp.maximum(jnp.dot(h1, w2_ref[:], preferred_element_type=f32)
                     + b2_ref[:], 0.0)
    adv = jnp.dot(h2, w3_ref[:], preferred_element_type=f32) + b3_ref[:]

    adv3 = adv.reshape(BB, SJ, 1)
    shap = jnp.sum(adv3 * gl, axis=1) * (1.0 / SAMPLE_SIZE)

    hv = jnp.maximum(jnp.dot(st, vw1_ref[:], preferred_element_type=f32)
                     + vb1_ref[:], 0.0)
    vs = jnp.sum(hv * vw2_ref[:], axis=1, keepdims=True) + vb2_ref[:]
    out_ref[:] = shap + vs


def kernel(states, agent_qs, W1, b1, W2, b2, W3, b3, VW1, Vb1, VW2, Vb2):
    f32 = jnp.float32
    bs0, t = states.shape[0], states.shape[1]

    u = jax.random.uniform(jax.random.key(42), (RTOT, N_AGENTS))
    pos = jnp.argsort(u, axis=1)
    perm = jnp.argsort(pos, axis=1).astype(jnp.int32)
    b_loc = ((jnp.arange(RTOT) // SAMPLE_SIZE) -
             (jnp.arange(RTOT) // RPW) * (RPW // SAMPLE_SIZE))[:, None]
    widx = ((b_loc * N_AGENTS + perm) * N_ACTIONS).astype(jnp.int32)
    widx = widx.reshape(RTOT * N_AGENTS)
    permp = perm.reshape(B_TOTAL, SJ)
    gl_const = (permp[:, :, None] ==
                jnp.arange(N_AGENTS)[None, None, :]).astype(f32)

    aq_flat = agent_qs.reshape(B_TOTAL * N_AGENTS * N_ACTIONS)
    pflat = _sc_gather(aq_flat, widx).reshape(B_TOTAL, SAMPLE_SIZE, SJ)

    states_r = states.reshape(B_TOTAL, STATE_DIM)
    w1s = W1[:, :STATE_DIM].T
    w1a = W1[:, STATE_DIM:].T
    row = lambda v: v.reshape(1, -1).astype(f32)
    const = lambda shape: pl.BlockSpec(shape, lambda i: tuple(0 for _ in shape))

    out = pl.pallas_call(
        _mixer_block,
        grid=(GRID,),
        in_specs=[
            pl.BlockSpec((BB, SAMPLE_SIZE, SJ), lambda i: (i, 0, 0)),
            pl.BlockSpec((BB, SJ, N_AGENTS), lambda i: (i, 0, 0)),
            pl.BlockSpec((BB, STATE_DIM), lambda i: (i, 0)),
            const((N_AGENTS, SJ)),
            const((ROWS, BB)),
            const((STATE_DIM, EMBED)),
            const((N_AGENTS * N_ACTIONS, EMBED)),
            const((1, EMBED)),
            const((EMBED, EMBED)),
            const((1, EMBED)),
            const((EMBED, 1)),
            const((1, 1)),
            const((STATE_DIM, EMBED)),
            const((1, EMBED)),
            const((1, EMBED)),
            const((1, 1)),
        ],
        out_specs=pl.BlockSpec((BB, N_AGENTS), lambda i: (i, 0)),
        out_shape=jax.ShapeDtypeStruct((B_TOTAL, N_AGENTS), f32),
    )(pflat, gl_const, states_r, _TRIL, _REPB, w1s, w1a, row(b1),
      W2.T, row(b2), W3.reshape(-1, 1).astype(f32),
      jnp.reshape(b3, (1, 1)).astype(f32), VW1.T, row(Vb1), row(VW2),
      jnp.reshape(Vb2, (1, 1)).astype(f32))
    return out.reshape(bs0, t, N_AGENTS)


# speedup vs baseline: 1.1690x; 1.1690x over previous
import jax
import jax.numpy as jnp
import numpy as np
from jax import lax
from jax.experimental import pallas as pl
from jax.experimental.pallas import tpu as pltpu
from jax.experimental.pallas import tpu_sc as plsc

N_AGENTS = 16
N_ACTIONS = 8
SAMPLE_SIZE = 8
STATE_DIM = 64
EMBED = 64
B_TOTAL = 1024
BB = 32
GRID = B_TOTAL // BB
R = BB * SAMPLE_SIZE
ROWS = R * N_AGENTS
SJ = SAMPLE_SIZE * N_AGENTS
RTOT = B_TOTAL * SAMPLE_SIZE
NW = 32
RPW = RTOT // NW
WORDS_PW = RPW * N_AGENTS
OUT_PW = RPW * 128

_TRIL = ((np.arange(SJ)[None, :] // N_ACTIONS) <=
         np.arange(N_AGENTS)[:, None]).astype(np.float32)
_REPB = ((np.arange(ROWS)[:, None] // SJ) ==
         np.arange(BB)[None, :]).astype(np.float32)


def _sc_gather_body(aq_hbm, idx_hbm, out_hbm, aq_v, idx_v, out_v):
    wid = lax.axis_index("s") * 2 + lax.axis_index("c")
    pltpu.sync_copy(aq_hbm.at[pl.ds(wid * WORDS_PW, WORDS_PW)], aq_v)
    pltpu.sync_copy(idx_hbm.at[pl.ds(wid * WORDS_PW, WORDS_PW)], idx_v)

    def body(r, carry):
        base = idx_v[pl.ds(r * N_AGENTS, N_AGENTS)]
        col = r * 128 + lax.iota(jnp.int32, N_AGENTS) * N_ACTIONS
        for c in range(N_ACTIONS):
            vals = plsc.load_gather(aq_v, (base + c,))
            plsc.store_scatter(out_v, (col + c,), vals)
        return carry

    lax.fori_loop(0, RPW, body, 0)
    pltpu.sync_copy(out_v, out_hbm.at[pl.ds(wid * OUT_PW, OUT_PW)])


def _sc_gather(aq_flat, widx):
    mesh = plsc.VectorSubcoreMesh(core_axis_name="c", subcore_axis_name="s",
                                  num_cores=2, num_subcores=16)
    fn = pl.kernel(
        _sc_gather_body,
        out_type=jax.ShapeDtypeStruct((RTOT * 128,), jnp.float32),
        mesh=mesh,
        compiler_params=pltpu.CompilerParams(needs_layout_passes=False),
        scratch_types=[
            pltpu.VMEM((WORDS_PW,), jnp.float32),
            pltpu.VMEM((WORDS_PW,), jnp.int32),
            pltpu.VMEM((OUT_PW,), jnp.float32),
        ],
    )
    return fn(aq_flat, widx)


def _mixer_block(pf_ref, gl_ref, st_ref, tril_ref, repb_ref,
                 w1s_ref, w1a_ref, b1_ref, w2_ref, b2_ref, w3_ref, b3_ref,
                 vw1_ref, vb1_ref, vw2_ref, vb2_ref, out_ref):
    f32 = jnp.float32
    pflat3 = pf_ref[:]
    gl = gl_ref[:]
    st = st_ref[:]

    xmask = (pflat3[:, :, None, :] *
             tril_ref[:][None, None]).reshape(ROWS, SJ)

    sp = jnp.dot(st, w1s_ref[:], preferred_element_type=f32)
    h1 = jnp.maximum(
        jnp.dot(xmask, w1a_ref[:], preferred_element_type=f32)
        + jnp.dot(repb_ref[:], sp, preferred_element_type=f32)
        + b1_ref[:], 0.0)
    h2 = jnp.maximum(jnp.dot(h1, w2_ref[:], preferred_element_type=f32)
                     + b2_ref[:], 0.0)
    adv = jnp.dot(h2, w3_ref[:], preferred_element_type=f32) + b3_ref[:]

    adv3 = adv.reshape(BB, SJ, 1)
    shap = jnp.sum(adv3 * gl, axis=1) * (1.0 / SAMPLE_SIZE)

    hv = jnp.maximum(jnp.dot(st, vw1_ref[:], preferred_element_type=f32)
                     + vb1_ref[:], 0.0)
    vs = jnp.sum(hv * vw2_ref[:], axis=1, keepdims=True) + vb2_ref[:]
    out_ref[:] = shap + vs


def kernel(states, agent_qs, W1, b1, W2, b2, W3, b3, VW1, Vb1, VW2, Vb2):
    f32 = jnp.float32
    bs0, t = states.shape[0], states.shape[1]

    u = jax.random.uniform(jax.random.key(42), (RTOT, N_AGENTS))
    pos = jnp.argsort(u, axis=1)
    perm = jnp.argsort(pos, axis=1).astype(jnp.int32)
    b_loc = ((jnp.arange(RTOT) // SAMPLE_SIZE) -
             (jnp.arange(RTOT) // RPW) * (RPW // SAMPLE_SIZE))[:, None]
    widx = ((b_loc * N_AGENTS + perm) * N_ACTIONS).astype(jnp.int32)
    widx = widx.reshape(RTOT * N_AGENTS)
    permp = perm.reshape(B_TOTAL, SJ)
    gl_const = (permp[:, :, None] ==
                jnp.arange(N_AGENTS)[None, None, :]).astype(f32)

    aq_flat = agent_qs.reshape(B_TOTAL * N_AGENTS * N_ACTIONS)
    pflat = _sc_gather(aq_flat, widx).reshape(B_TOTAL, SAMPLE_SIZE, SJ)

    states_r = states.reshape(B_TOTAL, STATE_DIM)
    w1s = W1[:, :STATE_DIM].T
    w1a = W1[:, STATE_DIM:].T
    row = lambda v: v.reshape(1, -1).astype(f32)
    const = lambda shape: pl.BlockSpec(shape, lambda i: tuple(0 for _ in shape))

    out = pl.pallas_call(
        _mixer_block,
        grid=(GRID,),
        in_specs=[
            pl.BlockSpec((BB, SAMPLE_SIZE, SJ), lambda i: (i, 0, 0)),
            pl.BlockSpec((BB, SJ, N_AGENTS), lambda i: (i, 0, 0)),
            pl.BlockSpec((BB, STATE_DIM), lambda i: (i, 0)),
            const((N_AGENTS, SJ)),
            const((ROWS, BB)),
            const((STATE_DIM, EMBED)),
            const((N_AGENTS * N_ACTIONS, EMBED)),
            const((1, EMBED)),
            const((EMBED, EMBED)),
            const((1, EMBED)),
            const((EMBED, 1)),
            const((1, 1)),
            const((STATE_DIM, EMBED)),
            const((1, EMBED)),
            const((1, EMBED)),
            const((1, 1)),
        ],
        out_specs=pl.BlockSpec((BB, N_AGENTS), lambda i: (i, 0)),
        out_shape=jax.ShapeDtypeStruct((B_TOTAL, N_AGENTS), f32),
    )(pflat, gl_const, states_r, _TRIL, _REPB, w1s, w1a, row(b1),
      W2.T, row(b2), W3.reshape(-1, 1).astype(f32),
      jnp.reshape(b3, (1, 1)).astype(f32), VW1.T, row(Vb1), row(VW2),
      jnp.reshape(Vb2, (1, 1)).astype(f32))
    return out.reshape(bs0, t, N_AGENTS)


# speedup vs baseline: 1.2159x; 1.0401x over previous
import jax
import jax.numpy as jnp
import numpy as np
from jax import lax
from jax.experimental import pallas as pl
from jax.experimental.pallas import tpu as pltpu
from jax.experimental.pallas import tpu_sc as plsc

N_AGENTS = 16
N_ACTIONS = 8
SAMPLE_SIZE = 8
STATE_DIM = 64
EMBED = 64
B_TOTAL = 1024
BB = 64
GRID = B_TOTAL // BB
R = BB * SAMPLE_SIZE
ROWS = R * N_AGENTS
SJ = SAMPLE_SIZE * N_AGENTS
RTOT = B_TOTAL * SAMPLE_SIZE
NW = 32
RPW = RTOT // NW
WORDS_PW = RPW * N_AGENTS
OUT_PW = RPW * 128

_TRIL = ((np.arange(SJ)[None, :] // N_ACTIONS) <=
         np.arange(N_AGENTS)[:, None]).astype(np.float32)
_REPB = ((np.arange(ROWS)[:, None] // SJ) ==
         np.arange(BB)[None, :]).astype(np.float32)


def _sc_gather_body(aq_hbm, idx_hbm, out_hbm, aq_v, idx_v, out_v):
    wid = lax.axis_index("s") * 2 + lax.axis_index("c")
    pltpu.sync_copy(aq_hbm.at[pl.ds(wid * WORDS_PW, WORDS_PW)], aq_v)
    pltpu.sync_copy(idx_hbm.at[pl.ds(wid * WORDS_PW, WORDS_PW)], idx_v)

    def body(r, carry):
        base = idx_v[pl.ds(r * N_AGENTS, N_AGENTS)]
        col = r * 128 + lax.iota(jnp.int32, N_AGENTS) * N_ACTIONS
        for c in range(N_ACTIONS):
            vals = plsc.load_gather(aq_v, (base + c,))
            plsc.store_scatter(out_v, (col + c,), vals)
        return carry

    lax.fori_loop(0, RPW, body, 0)
    pltpu.sync_copy(out_v, out_hbm.at[pl.ds(wid * OUT_PW, OUT_PW)])


def _sc_gather(aq_flat, widx):
    mesh = plsc.VectorSubcoreMesh(core_axis_name="c", subcore_axis_name="s",
                                  num_cores=2, num_subcores=16)
    fn = pl.kernel(
        _sc_gather_body,
        out_type=jax.ShapeDtypeStruct((RTOT * 128,), jnp.float32),
        mesh=mesh,
        compiler_params=pltpu.CompilerParams(needs_layout_passes=False),
        scratch_types=[
            pltpu.VMEM((WORDS_PW,), jnp.float32),
            pltpu.VMEM((WORDS_PW,), jnp.int32),
            pltpu.VMEM((OUT_PW,), jnp.float32),
        ],
    )
    return fn(aq_flat, widx)


def _mixer_block(pf_ref, gl_ref, st_ref, tril_ref, repb_ref,
                 w1s_ref, w1a_ref, b1_ref, w2_ref, b2_ref, w3_ref, b3_ref,
                 vw1_ref, vb1_ref, vw2_ref, vb2_ref, out_ref):
    f32 = jnp.float32
    pflat3 = pf_ref[:]
    gl = gl_ref[:]
    st = st_ref[:]

    xmask = (pflat3[:, :, None, :] *
             tril_ref[:][None, None]).reshape(ROWS, SJ)

    sp = jnp.dot(st, w1s_ref[:], preferred_element_type=f32)
    h1 = jnp.maximum(
        jnp.dot(xmask, w1a_ref[:], preferred_element_type=f32)
        + jnp.dot(repb_ref[:], sp, preferred_element_type=f32)
        + b1_ref[:], 0.0)
    h2 = jnp.maximum(jnp.dot(h1, w2_ref[:], preferred_element_type=f32)
                     + b2_ref[:], 0.0)
    adv = jnp.dot(h2, w3_ref[:], preferred_element_type=f32) + b3_ref[:]

    adv3 = adv.reshape(BB, SJ, 1)
    shap = jnp.sum(adv3 * gl, axis=1) * (1.0 / SAMPLE_SIZE)

    hv = jnp.maximum(jnp.dot(st, vw1_ref[:], preferred_element_type=f32)
                     + vb1_ref[:], 0.0)
    vs = jnp.sum(hv * vw2_ref[:], axis=1, keepdims=True) + vb2_ref[:]
    out_ref[:] = shap + vs


def kernel(states, agent_qs, W1, b1, W2, b2, W3, b3, VW1, Vb1, VW2, Vb2):
    f32 = jnp.float32
    bs0, t = states.shape[0], states.shape[1]

    u = jax.random.uniform(jax.random.key(42), (RTOT, N_AGENTS))
    pos = jnp.argsort(u, axis=1)
    perm = jnp.argsort(pos, axis=1).astype(jnp.int32)
    b_loc = ((jnp.arange(RTOT) // SAMPLE_SIZE) -
             (jnp.arange(RTOT) // RPW) * (RPW // SAMPLE_SIZE))[:, None]
    widx = ((b_loc * N_AGENTS + perm) * N_ACTIONS).astype(jnp.int32)
    widx = widx.reshape(RTOT * N_AGENTS)
    permp = perm.reshape(B_TOTAL, SJ)
    gl_const = (permp[:, :, None] ==
                jnp.arange(N_AGENTS)[None, None, :]).astype(f32)

    aq_flat = agent_qs.reshape(B_TOTAL * N_AGENTS * N_ACTIONS)
    pflat = _sc_gather(aq_flat, widx).reshape(B_TOTAL, SAMPLE_SIZE, SJ)

    states_r = states.reshape(B_TOTAL, STATE_DIM)
    w1s = W1[:, :STATE_DIM].T
    w1a = W1[:, STATE_DIM:].T
    row = lambda v: v.reshape(1, -1).astype(f32)
    const = lambda shape: pl.BlockSpec(shape, lambda i: tuple(0 for _ in shape))

    out = pl.pallas_call(
        _mixer_block,
        grid=(GRID,),
        in_specs=[
            pl.BlockSpec((BB, SAMPLE_SIZE, SJ), lambda i: (i, 0, 0)),
            pl.BlockSpec((BB, SJ, N_AGENTS), lambda i: (i, 0, 0)),
            pl.BlockSpec((BB, STATE_DIM), lambda i: (i, 0)),
            const((N_AGENTS, SJ)),
            const((ROWS, BB)),
            const((STATE_DIM, EMBED)),
            const((N_AGENTS * N_ACTIONS, EMBED)),
            const((1, EMBED)),
            const((EMBED, EMBED)),
            const((1, EMBED)),
            const((EMBED, 1)),
            const((1, 1)),
            const((STATE_DIM, EMBED)),
            const((1, EMBED)),
            const((1, EMBED)),
            const((1, 1)),
        ],
        out_specs=pl.BlockSpec((BB, N_AGENTS), lambda i: (i, 0)),
        out_shape=jax.ShapeDtypeStruct((B_TOTAL, N_AGENTS), f32),
    )(pflat, gl_const, states_r, _TRIL, _REPB, w1s, w1a, row(b1),
      W2.T, row(b2), W3.reshape(-1, 1).astype(f32),
      jnp.reshape(b3, (1, 1)).astype(f32), VW1.T, row(Vb1), row(VW2),
      jnp.reshape(Vb2, (1, 1)).astype(f32))
    return out.reshape(bs0, t, N_AGENTS)


# speedup vs baseline: 1.4441x; 1.1877x over previous
import jax
import jax.numpy as jnp
import numpy as np
from jax import lax
from jax.experimental import pallas as pl
from jax.experimental.pallas import tpu as pltpu
from jax.experimental.pallas import tpu_sc as plsc

N_AGENTS = 16
N_ACTIONS = 8
SAMPLE_SIZE = 8
STATE_DIM = 64
EMBED = 64
B_TOTAL = 1024
BB = 64
GRID = B_TOTAL // BB
R = BB * SAMPLE_SIZE
ROWS = R * N_AGENTS
SJ = SAMPLE_SIZE * N_AGENTS
RTOT = B_TOTAL * SAMPLE_SIZE
NW = 32
RPW = RTOT // NW
WORDS_PW = RPW * N_AGENTS
OUT_PW = RPW * 128

_TRIL = ((np.arange(SJ)[None, :] // N_ACTIONS) <=
         np.arange(N_AGENTS)[:, None]).astype(np.float32)


def _sc_gather_body(aq_hbm, idx_hbm, out_hbm, aq_v, idx_v, out_v):
    wid = lax.axis_index("s") * 2 + lax.axis_index("c")
    pltpu.sync_copy(aq_hbm.at[pl.ds(wid * WORDS_PW, WORDS_PW)], aq_v)
    pltpu.sync_copy(idx_hbm.at[pl.ds(wid * WORDS_PW, WORDS_PW)], idx_v)

    def body(r, carry):
        base = idx_v[pl.ds(r * N_AGENTS, N_AGENTS)]
        col = r * 128 + lax.iota(jnp.int32, N_AGENTS) * N_ACTIONS
        for c in range(N_ACTIONS):
            vals = plsc.load_gather(aq_v, (base + c,))
            plsc.store_scatter(out_v, (col + c,), vals)
        return carry

    lax.fori_loop(0, RPW, body, 0)
    pltpu.sync_copy(out_v, out_hbm.at[pl.ds(wid * OUT_PW, OUT_PW)])


def _sc_gather(aq_flat, widx):
    mesh = plsc.VectorSubcoreMesh(core_axis_name="c", subcore_axis_name="s",
                                  num_cores=2, num_subcores=16)
    fn = pl.kernel(
        _sc_gather_body,
        out_type=jax.ShapeDtypeStruct((RTOT * 128,), jnp.float32),
        mesh=mesh,
        compiler_params=pltpu.CompilerParams(needs_layout_passes=False),
        scratch_types=[
            pltpu.VMEM((WORDS_PW,), jnp.float32),
            pltpu.VMEM((WORDS_PW,), jnp.int32),
            pltpu.VMEM((OUT_PW,), jnp.float32),
        ],
    )
    return fn(aq_flat, widx)


def _mixer_block(pf_ref, gl_ref, st_ref, tril_ref,
                 w1s_ref, w1a_ref, b1_ref, w2_ref, b2_ref, w3_ref, b3_ref,
                 vw1_ref, vb1_ref, vw2_ref, vb2_ref, out_ref):
    f32 = jnp.float32
    pflat3 = pf_ref[:]
    gl = gl_ref[:]
    st = st_ref[:]

    xmask = (pflat3[:, :, None, :] *
             tril_ref[:][None, None]).reshape(ROWS, SJ)

    sp = jnp.dot(st, w1s_ref[:], preferred_element_type=f32)
    hpre = jnp.dot(xmask, w1a_ref[:], preferred_element_type=f32)
    h1 = jnp.maximum(
        hpre.reshape(BB, SAMPLE_SIZE, N_AGENTS, EMBED)
        + sp[:, None, None, :] + b1_ref[:], 0.0).reshape(ROWS, EMBED)
    h2 = jnp.maximum(jnp.dot(h1, w2_ref[:], preferred_element_type=f32)
                     + b2_ref[:], 0.0)

    tt = lax.dot_general(h2.reshape(BB, SJ, EMBED), gl,
                         (((1,), (1,)), ((0,), (0,))),
                         preferred_element_type=f32)
    shap = (jnp.sum(tt * w3_ref[:][None], axis=1)
            * (1.0 / SAMPLE_SIZE) + b3_ref[:])

    hv = jnp.maximum(jnp.dot(st, vw1_ref[:], preferred_element_type=f32)
                     + vb1_ref[:], 0.0)
    vs = jnp.sum(hv * vw2_ref[:], axis=1, keepdims=True) + vb2_ref[:]
    out_ref[:] = shap + vs


def kernel(states, agent_qs, W1, b1, W2, b2, W3, b3, VW1, Vb1, VW2, Vb2):
    f32 = jnp.float32
    bs0, t = states.shape[0], states.shape[1]

    u = jax.random.uniform(jax.random.key(42), (RTOT, N_AGENTS))
    pos = jnp.argsort(u, axis=1)
    perm = jnp.argsort(pos, axis=1).astype(jnp.int32)
    b_loc = ((jnp.arange(RTOT) // SAMPLE_SIZE) -
             (jnp.arange(RTOT) // RPW) * (RPW // SAMPLE_SIZE))[:, None]
    widx = ((b_loc * N_AGENTS + perm) * N_ACTIONS).astype(jnp.int32)
    widx = widx.reshape(RTOT * N_AGENTS)
    permp = perm.reshape(B_TOTAL, SJ)
    gl_const = (permp[:, :, None] ==
                jnp.arange(N_AGENTS)[None, None, :]).astype(f32)

    aq_flat = agent_qs.reshape(B_TOTAL * N_AGENTS * N_ACTIONS)
    pflat = _sc_gather(aq_flat, widx).reshape(B_TOTAL, SAMPLE_SIZE, SJ)

    states_r = states.reshape(B_TOTAL, STATE_DIM)
    row = lambda v: v.reshape(1, -1).astype(f32)
    const = lambda shape: pl.BlockSpec(shape, lambda i: tuple(0 for _ in shape))

    out = pl.pallas_call(
        _mixer_block,
        grid=(GRID,),
        in_specs=[
            pl.BlockSpec((BB, SAMPLE_SIZE, SJ), lambda i: (i, 0, 0)),
            pl.BlockSpec((BB, SJ, N_AGENTS), lambda i: (i, 0, 0)),
            pl.BlockSpec((BB, STATE_DIM), lambda i: (i, 0)),
            const((N_AGENTS, SJ)),
            const((STATE_DIM, EMBED)),
            const((N_AGENTS * N_ACTIONS, EMBED)),
            const((1, EMBED)),
            const((EMBED, EMBED)),
            const((1, EMBED)),
            const((EMBED, 1)),
            const((1, 1)),
            const((STATE_DIM, EMBED)),
            const((1, EMBED)),
            const((1, EMBED)),
            const((1, 1)),
        ],
        out_specs=pl.BlockSpec((BB, N_AGENTS), lambda i: (i, 0)),
        out_shape=jax.ShapeDtypeStruct((B_TOTAL, N_AGENTS), f32),
    )(pflat, gl_const, states_r, _TRIL, W1[:, :STATE_DIM].T,
      W1[:, STATE_DIM:].T, row(b1),
      W2.T, row(b2), W3.reshape(-1, 1).astype(f32),
      jnp.reshape(b3, (1, 1)).astype(f32), VW1.T, row(Vb1), row(VW2),
      jnp.reshape(Vb2, (1, 1)).astype(f32))
    return out.reshape(bs0, t, N_AGENTS)


# speedup vs baseline: 1.4564x; 1.0085x over previous
import jax
import jax.numpy as jnp
import numpy as np
from jax import lax
from jax.experimental import pallas as pl
from jax.experimental.pallas import tpu as pltpu
from jax.experimental.pallas import tpu_sc as plsc

N_AGENTS = 16
N_ACTIONS = 8
SAMPLE_SIZE = 8
STATE_DIM = 64
EMBED = 64
B_TOTAL = 1024
BB = 128
GRID = B_TOTAL // BB
R = BB * SAMPLE_SIZE
ROWS = R * N_AGENTS
SJ = SAMPLE_SIZE * N_AGENTS
RTOT = B_TOTAL * SAMPLE_SIZE
NW = 32
RPW = RTOT // NW
WORDS_PW = RPW * N_AGENTS
OUT_PW = RPW * 128

_TRIL = ((np.arange(SJ)[None, :] // N_ACTIONS) <=
         np.arange(N_AGENTS)[:, None]).astype(np.float32)


def _sc_gather_body(aq_hbm, idx_hbm, out_hbm, aq_v, idx_v, out_v):
    wid = lax.axis_index("s") * 2 + lax.axis_index("c")
    pltpu.sync_copy(aq_hbm.at[pl.ds(wid * WORDS_PW, WORDS_PW)], aq_v)
    pltpu.sync_copy(idx_hbm.at[pl.ds(wid * WORDS_PW, WORDS_PW)], idx_v)

    def body(r, carry):
        base = idx_v[pl.ds(r * N_AGENTS, N_AGENTS)]
        col = r * 128 + lax.iota(jnp.int32, N_AGENTS) * N_ACTIONS
        for c in range(N_ACTIONS):
            vals = plsc.load_gather(aq_v, (base + c,))
            plsc.store_scatter(out_v, (col + c,), vals)
        return carry

    lax.fori_loop(0, RPW, body, 0)
    pltpu.sync_copy(out_v, out_hbm.at[pl.ds(wid * OUT_PW, OUT_PW)])


def _sc_gather(aq_flat, widx):
    mesh = plsc.VectorSubcoreMesh(core_axis_name="c", subcore_axis_name="s",
                                  num_cores=2, num_subcores=16)
    fn = pl.kernel(
        _sc_gather_body,
        out_type=jax.ShapeDtypeStruct((RTOT * 128,), jnp.float32),
        mesh=mesh,
        compiler_params=pltpu.CompilerParams(needs_layout_passes=False),
        scratch_types=[
            pltpu.VMEM((WORDS_PW,), jnp.float32),
            pltpu.VMEM((WORDS_PW,), jnp.int32),
            pltpu.VMEM((OUT_PW,), jnp.float32),
        ],
    )
    return fn(aq_flat, widx)


def _mixer_block(pf_ref, gl_ref, st_ref, tril_ref,
                 w1s_ref, w1a_ref, b1_ref, w2_ref, b2_ref, w3_ref, b3_ref,
                 vw1_ref, vb1_ref, vw2_ref, vb2_ref, out_ref):
    f32 = jnp.float32
    pflat3 = pf_ref[:]
    gl = gl_ref[:]
    st = st_ref[:]

    xmask = (pflat3[:, :, None, :] *
             tril_ref[:][None, None]).reshape(ROWS, SJ)

    sp = jnp.dot(st, w1s_ref[:], preferred_element_type=f32)
    hpre = jnp.dot(xmask, w1a_ref[:], preferred_element_type=f32)
    h1 = jnp.maximum(
        hpre.reshape(BB, SAMPLE_SIZE, N_AGENTS, EMBED)
        + sp[:, None, None, :] + b1_ref[:], 0.0).reshape(ROWS, EMBED)
    h2 = jnp.maximum(jnp.dot(h1, w2_ref[:], preferred_element_type=f32)
                     + b2_ref[:], 0.0)

    tt = lax.dot_general(h2.reshape(BB, SJ, EMBED), gl,
                         (((1,), (1,)), ((0,), (0,))),
                         preferred_element_type=f32)
    shap = (jnp.sum(tt * w3_ref[:][None], axis=1)
            * (1.0 / SAMPLE_SIZE) + b3_ref[:])

    hv = jnp.maximum(jnp.dot(st, vw1_ref[:], preferred_element_type=f32)
                     + vb1_ref[:], 0.0)
    vs = jnp.sum(hv * vw2_ref[:], axis=1, keepdims=True) + vb2_ref[:]
    out_ref[:] = shap + vs


def kernel(states, agent_qs, W1, b1, W2, b2, W3, b3, VW1, Vb1, VW2, Vb2):
    f32 = jnp.float32
    bs0, t = states.shape[0], states.shape[1]

    u = jax.random.uniform(jax.random.key(42), (RTOT, N_AGENTS))
    pos = jnp.argsort(u, axis=1)
    perm = jnp.argsort(pos, axis=1).astype(jnp.int32)
    b_loc = ((jnp.arange(RTOT) // SAMPLE_SIZE) -
             (jnp.arange(RTOT) // RPW) * (RPW // SAMPLE_SIZE))[:, None]
    widx = ((b_loc * N_AGENTS + perm) * N_ACTIONS).astype(jnp.int32)
    widx = widx.reshape(RTOT * N_AGENTS)
    permp = perm.reshape(B_TOTAL, SJ)
    gl_const = (permp[:, :, None] ==
                jnp.arange(N_AGENTS)[None, None, :]).astype(f32)

    aq_flat = agent_qs.reshape(B_TOTAL * N_AGENTS * N_ACTIONS)
    pflat = _sc_gather(aq_flat, widx).reshape(B_TOTAL, SAMPLE_SIZE, SJ)

    states_r = states.reshape(B_TOTAL, STATE_DIM)
    row = lambda v: v.reshape(1, -1).astype(f32)
    const = lambda shape: pl.BlockSpec(shape, lambda i: tuple(0 for _ in shape))

    out = pl.pallas_call(
        _mixer_block,
        grid=(GRID,),
        in_specs=[
            pl.BlockSpec((BB, SAMPLE_SIZE, SJ), lambda i: (i, 0, 0)),
            pl.BlockSpec((BB, SJ, N_AGENTS), lambda i: (i, 0, 0)),
            pl.BlockSpec((BB, STATE_DIM), lambda i: (i, 0)),
            const((N_AGENTS, SJ)),
            const((STATE_DIM, EMBED)),
            const((N_AGENTS * N_ACTIONS, EMBED)),
            const((1, EMBED)),
            const((EMBED, EMBED)),
            const((1, EMBED)),
            const((EMBED, 1)),
            const((1, 1)),
            const((STATE_DIM, EMBED)),
            const((1, EMBED)),
            const((1, EMBED)),
            const((1, 1)),
        ],
        out_specs=pl.BlockSpec((BB, N_AGENTS), lambda i: (i, 0)),
        out_shape=jax.ShapeDtypeStruct((B_TOTAL, N_AGENTS), f32),
    )(pflat, gl_const, states_r, _TRIL, W1[:, :STATE_DIM].T,
      W1[:, STATE_DIM:].T, row(b1),
      W2.T, row(b2), W3.reshape(-1, 1).astype(f32),
      jnp.reshape(b3, (1, 1)).astype(f32), VW1.T, row(Vb1), row(VW2),
      jnp.reshape(Vb2, (1, 1)).astype(f32))
    return out.reshape(bs0, t, N_AGENTS)
